# Initial kernel scaffold; baseline (speedup 1.0000x reference)
import jax
import jax.numpy as jnp
from jax.experimental import pallas as pl


def kernel(x_nchw, conv0_w_mat, conv0_bias, conv0_w_group, conv0_b_group, conv1_w_mat, conv1_bias, conv1_w_group, conv1_b_group, conv2_w_mat, conv2_bias, conv2_w_group, conv2_b_group, conv3_w_mat, conv3_bias, conv3_w_group, conv3_b_group, conv4_w_mat, conv4_bias, conv4_w_group, conv4_b_group, conv5_w_mat, conv5_bias, conv5_w_group, conv5_b_group, conv6_w_mat, conv6_bias, conv6_w_group, conv6_b_group, conv7_w_mat, conv7_bias, conv7_w_group, conv7_b_group, conv8_w_mat, conv8_bias, conv8_w_group, conv8_b_group, conv9_w_mat, conv9_bias, conv9_w_group, conv9_b_group, fc_wt, fc_b):
    raise NotImplementedError("write your pallas kernel here")


# speedup vs baseline: 2.0465x; 2.0465x over previous
import jax
import jax.numpy as jnp
from jax.experimental import pallas as pl
from jax.experimental.pallas import tpu as pltpu

_VMEM_LIMIT = 64 * 1024 * 1024


def _make_stage_kernel(H, W, cp):
    r = 128 // cp
    Mr = H * (W // r)
    W_LANES = W * cp

    def body(x_ref, wa_ref, ba_ref, wb_ref, bb_ref, o_ref):
        x = x_ref[0]
        for w_ref, b_ref in ((wa_ref, ba_ref), (wb_ref, bb_ref)):
            xp = jnp.pad(x, ((1, 1), (cp, cp)))
            taps = []
            for dy in range(3):
                for dx in range(3):
                    t = xp[dy:dy + H, dx * cp:dx * cp + W_LANES]
                    taps.append(t.reshape(Mr, 128))
            p = jnp.concatenate(taps, axis=1)
            y = jnp.dot(p, w_ref[...], preferred_element_type=jnp.float32)
            y = jnp.maximum(y + b_ref[...], 0.0)
            x = y.reshape(H, W_LANES)
        o_ref[0] = x

    return body


def _stage_call(x3, wa, ba, wb, bb, H, W, cp):
    N = x3.shape[0]
    W_LANES = W * cp
    return pl.pallas_call(
        _make_stage_kernel(H, W, cp),
        out_shape=jax.ShapeDtypeStruct((N, H, W_LANES), jnp.float32),
        grid_spec=pltpu.PrefetchScalarGridSpec(
            num_scalar_prefetch=0,
            grid=(N,),
            in_specs=[
                pl.BlockSpec((1, H, W_LANES), lambda i: (i, 0, 0)),
                pl.BlockSpec((1152, 128), lambda i: (0, 0)),
                pl.BlockSpec((1, 128), lambda i: (0, 0)),
                pl.BlockSpec((1152, 128), lambda i: (0, 0)),
                pl.BlockSpec((1, 128), lambda i: (0, 0)),
            ],
            out_specs=pl.BlockSpec((1, H, W_LANES), lambda i: (i, 0, 0)),
        ),
        compiler_params=pltpu.CompilerParams(
            dimension_semantics=("parallel",),
            vmem_limit_bytes=_VMEM_LIMIT),
    )(x3, wa, ba, wb, bb)


def _prep_weight(w_group, cin, cp, r):
    w = w_group.reshape(r, 9, cin, 128)
    w = jnp.transpose(w, (1, 0, 2, 3))
    if cin < cp:
        w = jnp.pad(w, ((0, 0), (0, 0), (0, cp - cin), (0, 0)))
    return w.reshape(9 * r * cp, 128)


def _fc_kernel(x_ref, w_ref, b_ref, o_ref, acc_ref):
    k = pl.program_id(1)

    @pl.when(k == 0)
    def _():
        acc_ref[...] = jnp.zeros_like(acc_ref)

    acc_ref[...] += jnp.dot(x_ref[...], w_ref[...],
                            preferred_element_type=jnp.float32)

    @pl.when(k == pl.num_programs(1) - 1)
    def _():
        o_ref[...] = (acc_ref[...] + b_ref[...]).astype(o_ref.dtype)


def kernel(x_nchw, conv0_w_mat, conv0_bias, conv0_w_group, conv0_b_group, conv1_w_mat, conv1_bias, conv1_w_group, conv1_b_group, conv2_w_mat, conv2_bias, conv2_w_group, conv2_b_group, conv3_w_mat, conv3_bias, conv3_w_group, conv3_b_group, conv4_w_mat, conv4_bias, conv4_w_group, conv4_b_group, conv5_w_mat, conv5_bias, conv5_w_group, conv5_b_group, conv6_w_mat, conv6_bias, conv6_w_group, conv6_b_group, conv7_w_mat, conv7_bias, conv7_w_group, conv7_b_group, conv8_w_mat, conv8_bias, conv8_w_group, conv8_b_group, conv9_w_mat, conv9_bias, conv9_w_group, conv9_b_group, fc_wt, fc_b):
    w_mats = [conv0_w_mat, conv1_w_mat, conv2_w_mat, conv3_w_mat,
              conv4_w_mat, conv5_w_mat, conv6_w_mat, conv7_w_mat,
              conv8_w_mat, conv9_w_mat]
    w_groups = [conv0_w_group, conv1_w_group, conv2_w_group, conv3_w_group,
                conv4_w_group, conv5_w_group, conv6_w_group, conv7_w_group,
                conv8_w_group, conv9_w_group]
    b_groups = [conv0_b_group, conv1_b_group, conv2_b_group, conv3_b_group,
                conv4_b_group, conv5_b_group, conv6_b_group, conv7_b_group,
                conv8_b_group, conv9_b_group]

    N, cin0, H0, W0 = x_nchw.shape
    x = jnp.transpose(x_nchw, (0, 2, 3, 1)).astype(jnp.float32)

    H, W = H0, W0
    for s in range(5):
        la, lb = 2 * s, 2 * s + 1
        Ka, cs = w_mats[la].shape
        cin = Ka // 9
        r = 128 // cs
        if s == 0:
            x = jnp.pad(x, ((0, 0), (0, 0), (0, 0), (0, cs - cin)))
        wa = _prep_weight(w_groups[la], cin, cs, r)
        wb = _prep_weight(w_groups[lb], cs, cs, r)
        x3 = x.reshape(N, H, W * cs)
        y3 = _stage_call(x3, wa, b_groups[la].reshape(1, 128),
                         wb, b_groups[lb].reshape(1, 128), H, W, cs)
        x = y3.reshape(N, H, W, cs)
        if s < 4:
            x = jnp.maximum(jnp.maximum(x[:, 0::2, 0::2, :], x[:, 0::2, 1::2, :]),
                            jnp.maximum(x[:, 1::2, 0::2, :], x[:, 1::2, 1::2, :]))
            H, W = H // 2, W // 2
            _, cs_next = w_mats[la + 2].shape
            x = jnp.pad(x, ((0, 0), (0, 0), (0, 0), (0, cs_next - cs)))

    x_flat = x.transpose(0, 3, 1, 2).reshape(N, -1)

    Kfc, Nfc = fc_wt.shape
    tn = 512 if Nfc % 512 == 0 else Nfc
    tk = 1664 if Kfc % 1664 == 0 else Kfc
    out = pl.pallas_call(
        _fc_kernel,
        out_shape=jax.ShapeDtypeStruct((N, Nfc), jnp.float32),
        grid_spec=pltpu.PrefetchScalarGridSpec(
            num_scalar_prefetch=0,
            grid=(Nfc // tn, Kfc // tk),
            in_specs=[
                pl.BlockSpec((N, tk), lambda j, k: (0, k)),
                pl.BlockSpec((tk, tn), lambda j, k: (k, j)),
                pl.BlockSpec((1, tn), lambda j, k: (0, j)),
            ],
            out_specs=pl.BlockSpec((N, tn), lambda j, k: (0, j)),
            scratch_shapes=[pltpu.VMEM((N, tn), jnp.float32)],
        ),
        compiler_params=pltpu.CompilerParams(
            dimension_semantics=("parallel", "arbitrary"),
            vmem_limit_bytes=_VMEM_LIMIT),
    )(x_flat, fc_wt, fc_b.reshape(1, Nfc))
    return out


# speedup vs baseline: 14.3757x; 7.0246x over previous
import jax
import jax.numpy as jnp
from jax.experimental import pallas as pl
from jax.experimental.pallas import tpu as pltpu

_VMEM_LIMIT = 96 * 1024 * 1024
_HI = jax.lax.Precision.HIGHEST


def _conv(x, w, b, H, W, cp):
    r = 128 // cp
    Mr = H * (W // r)
    WL = W * cp
    xp = jnp.pad(x, ((1, 1), (cp, cp)))
    taps = []
    for dy in range(3):
        for dx in range(3):
            t = xp[dy:dy + H, dx * cp:dx * cp + WL]
            taps.append(t.reshape(Mr, 128))
    p = jnp.concatenate(taps, axis=1)
    y = jnp.dot(p, w, preferred_element_type=jnp.float32)
    y = jnp.maximum(y + b, 0.0)
    return y.reshape(H, WL)


def _pool(x, H, W, cp):
    WL = W * cp
    t = x.reshape(H // 2, 2 * WL)
    m = jnp.maximum(t[:, :WL], t[:, WL:])
    mc = m.reshape((H // 2) * (W * cp // 128), 128)
    li = jax.lax.broadcasted_iota(jnp.int32, (128, 128), 0)
    lo = jax.lax.broadcasted_iota(jnp.int32, (128, 128), 1)
    q, c = lo // (2 * cp), lo % (2 * cp)
    valid = c < cp
    pe = (valid & (li == 2 * q * cp + c)).astype(jnp.float32)
    po = (valid & (li == (2 * q + 1) * cp + c)).astype(jnp.float32)
    y = jnp.maximum(
        jnp.dot(mc, pe, preferred_element_type=jnp.float32, precision=_HI),
        jnp.dot(mc, po, preferred_element_type=jnp.float32, precision=_HI))
    return y.reshape(H // 2, WL)


def _interleave_in(x, H, W, cp):
    cin = x.shape[0]
    li = jax.lax.broadcasted_iota(jnp.int32, (W, W * cp), 0)
    lo = jax.lax.broadcasted_iota(jnp.int32, (W, W * cp), 1)
    acc = None
    for ch in range(cin):
        s = ((lo // cp == li) & (lo % cp == ch)).astype(jnp.float32)
        d = jnp.dot(x[ch], s, preferred_element_type=jnp.float32,
                    precision=_HI)
        acc = d if acc is None else acc + d
    return acc


def _make_trunk_kernel(stage_dims, cin0):
    def body(x_ref, *refs):
        o_ref = refs[-1]
        H0, W0, cp0 = stage_dims[0]
        x = _interleave_in(x_ref[0], H0, W0, cp0)
        for s, (H, W, cp) in enumerate(stage_dims):
            x = _conv(x, refs[4 * s][...], refs[4 * s + 1][...], H, W, cp)
            x = _conv(x, refs[4 * s + 2][...], refs[4 * s + 3][...], H, W, cp)
            if s < 4:
                x = _pool(x, H, W, cp)
        Hf, Wf, _ = stage_dims[-1]
        t = x.reshape(Hf * Wf, 128)
        eye = jnp.eye(128, dtype=jnp.float32)
        o_ref[0] = jax.lax.dot_general(
            eye, t, (((1,), (1,)), ((), ())),
            preferred_element_type=jnp.float32, precision=_HI)
    return body


def _prep_weight(w_group, cin, cp, r):
    w = w_group.reshape(r, 9, cin, 128)
    w = jnp.transpose(w, (1, 0, 2, 3))
    if cin < cp:
        w = jnp.pad(w, ((0, 0), (0, 0), (0, cp - cin), (0, 0)))
    return w.reshape(9 * r * cp, 128)


def _fc_kernel(x_ref, w_ref, b_ref, o_ref, acc_ref):
    k = pl.program_id(1)

    @pl.when(k == 0)
    def _():
        acc_ref[...] = jnp.zeros_like(acc_ref)

    acc_ref[...] += jnp.dot(x_ref[...], w_ref[...],
                            preferred_element_type=jnp.float32)

    @pl.when(k == pl.num_programs(1) - 1)
    def _():
        o_ref[...] = (acc_ref[...] + b_ref[...]).astype(o_ref.dtype)


def kernel(x_nchw, conv0_w_mat, conv0_bias, conv0_w_group, conv0_b_group, conv1_w_mat, conv1_bias, conv1_w_group, conv1_b_group, conv2_w_mat, conv2_bias, conv2_w_group, conv2_b_group, conv3_w_mat, conv3_bias, conv3_w_group, conv3_b_group, conv4_w_mat, conv4_bias, conv4_w_group, conv4_b_group, conv5_w_mat, conv5_bias, conv5_w_group, conv5_b_group, conv6_w_mat, conv6_bias, conv6_w_group, conv6_b_group, conv7_w_mat, conv7_bias, conv7_w_group, conv7_b_group, conv8_w_mat, conv8_bias, conv8_w_group, conv8_b_group, conv9_w_mat, conv9_bias, conv9_w_group, conv9_b_group, fc_wt, fc_b):
    w_mats = [conv0_w_mat, conv1_w_mat, conv2_w_mat, conv3_w_mat,
              conv4_w_mat, conv5_w_mat, conv6_w_mat, conv7_w_mat,
              conv8_w_mat, conv9_w_mat]
    w_groups = [conv0_w_group, conv1_w_group, conv2_w_group, conv3_w_group,
                conv4_w_group, conv5_w_group, conv6_w_group, conv7_w_group,
                conv8_w_group, conv9_w_group]
    b_groups = [conv0_b_group, conv1_b_group, conv2_b_group, conv3_b_group,
                conv4_b_group, conv5_b_group, conv6_b_group, conv7_b_group,
                conv8_b_group, conv9_b_group]

    N, cin0, H0, W0 = x_nchw.shape

    stage_dims = []
    ops = []
    H, W = H0, W0
    prev_c = cin0
    for s in range(5):
        la, lb = 2 * s, 2 * s + 1
        if s > 0:
            H, W = H // 2, W // 2
        Ka, cs = w_mats[la].shape
        cin = Ka // 9
        assert cin == prev_c
        r = 128 // cs
        stage_dims.append((H, W, cs))
        ops.append(_prep_weight(w_groups[la], cin, cs, r))
        ops.append(b_groups[la].reshape(1, 128))
        ops.append(_prep_weight(w_groups[lb], cs, cs, r))
        ops.append(b_groups[lb].reshape(1, 128))
        prev_c = cs

    Hf, Wf, cf = stage_dims[-1]
    in_specs = [pl.BlockSpec((1, cin0, H0, W0), lambda i: (i, 0, 0, 0))]
    for s in range(5):
        in_specs.append(pl.BlockSpec((9 * 128, 128), lambda i: (0, 0)))
        in_specs.append(pl.BlockSpec((1, 128), lambda i: (0, 0)))
        in_specs.append(pl.BlockSpec((9 * 128, 128), lambda i: (0, 0)))
        in_specs.append(pl.BlockSpec((1, 128), lambda i: (0, 0)))

    feat = pl.pallas_call(
        _make_trunk_kernel(stage_dims, cin0),
        out_shape=jax.ShapeDtypeStruct((N, cf, Hf * Wf), jnp.float32),
        grid_spec=pltpu.PrefetchScalarGridSpec(
            num_scalar_prefetch=0,
            grid=(N,),
            in_specs=in_specs,
            out_specs=pl.BlockSpec((1, cf, Hf * Wf), lambda i: (i, 0, 0)),
        ),
        compiler_params=pltpu.CompilerParams(
            dimension_semantics=("parallel",),
            vmem_limit_bytes=_VMEM_LIMIT),
    )(x_nchw.astype(jnp.float32), *ops)

    x_flat = feat.reshape(N, cf * Hf * Wf)

    Kfc, Nfc = fc_wt.shape
    tn = 512 if Nfc % 512 == 0 else Nfc
    tk = 1664 if Kfc % 1664 == 0 else Kfc
    out = pl.pallas_call(
        _fc_kernel,
        out_shape=jax.ShapeDtypeStruct((N, Nfc), jnp.float32),
        grid_spec=pltpu.PrefetchScalarGridSpec(
            num_scalar_prefetch=0,
            grid=(Nfc // tn, Kfc // tk),
            in_specs=[
                pl.BlockSpec((N, tk), lambda j, k: (0, k)),
                pl.BlockSpec((tk, tn), lambda j, k: (k, j)),
                pl.BlockSpec((1, tn), lambda j, k: (0, j)),
            ],
            out_specs=pl.BlockSpec((N, tn), lambda j, k: (0, j)),
            scratch_shapes=[pltpu.VMEM((N, tn), jnp.float32)],
        ),
        compiler_params=pltpu.CompilerParams(
            dimension_semantics=("parallel", "arbitrary"),
            vmem_limit_bytes=_VMEM_LIMIT),
    )(x_flat, fc_wt, fc_b.reshape(1, Nfc))
    return out


# speedup vs baseline: 32.8858x; 2.2876x over previous
import jax
import jax.numpy as jnp
from jax.experimental import pallas as pl
from jax.experimental.pallas import tpu as pltpu

_VMEM_LIMIT = 96 * 1024 * 1024


def _split_dot(x, s_bf16):
    hi = x.astype(jnp.bfloat16)
    lo = (x - hi.astype(jnp.float32)).astype(jnp.bfloat16)
    return (jnp.dot(hi, s_bf16, preferred_element_type=jnp.float32)
            + jnp.dot(lo, s_bf16, preferred_element_type=jnp.float32))


def _conv(x, w, b, Mr, chunks):
    rows = jax.lax.broadcasted_iota(jnp.int32, (Mr, 1), 0) % chunks
    not_first = rows != 0
    not_last = rows != (chunks - 1)
    z = jnp.zeros_like(x)
    taps = [
        jnp.pad(x[:-chunks], ((chunks, 0), (0, 0))),
        x,
        jnp.pad(x[chunks:], ((0, chunks), (0, 0))),
    ]
    y = None
    for dy in range(3):
        a = taps[dy]
        p = jnp.where(not_first, jnp.pad(a[:-1], ((1, 0), (0, 0))), z)
        q = jnp.where(not_last, jnp.pad(a[1:], ((0, 1), (0, 0))), z)
        d = (jnp.dot(a, w[3 * dy], preferred_element_type=jnp.float32)
             + jnp.dot(p, w[3 * dy + 1], preferred_element_type=jnp.float32)
             + jnp.dot(q, w[3 * dy + 2], preferred_element_type=jnp.float32))
        y = d if y is None else y + d
    return jnp.maximum(y + b, 0.0)


def _pool(x, H, W, cp, pe, po):
    WL = W * cp
    t = x.reshape(H, WL).reshape(H // 2, 2 * WL)
    m = jnp.maximum(t[:, :WL], t[:, WL:])
    mc = m.reshape((H // 2) * (WL // 128), 128)
    return jnp.maximum(_split_dot(mc, pe), _split_dot(mc, po))


def _make_trunk_kernel(stage_dims, cin0):
    def body(x_ref, s_ref, pp_ref, *refs):
        o_ref = refs[-1]
        H0, W0, cp0 = stage_dims[0]
        acc = None
        for ch in range(cin0):
            xi = x_ref[0][ch]
            hi = xi.astype(jnp.bfloat16)
            lo = (xi - hi.astype(jnp.float32)).astype(jnp.bfloat16)
            d = (jnp.dot(hi, s_ref[ch], preferred_element_type=jnp.float32)
                 + jnp.dot(lo, s_ref[ch], preferred_element_type=jnp.float32))
            acc = d if acc is None else acc + d
        chunks = W0 * cp0 // 128
        x = acc.reshape(H0 * chunks, 128)
        for s, (H, W, cp) in enumerate(stage_dims):
            Mr = H * chunks
            x = _conv(x, refs[4 * s][...], refs[4 * s + 1][...], Mr, chunks)
            x = _conv(x, refs[4 * s + 2][...], refs[4 * s + 3][...], Mr, chunks)
            if s < 4:
                x = _pool(x, H, W, cp, pp_ref[2 * s], pp_ref[2 * s + 1])
        eye = (jax.lax.broadcasted_iota(jnp.int32, (128, 128), 0)
               == jax.lax.broadcasted_iota(jnp.int32, (128, 128), 1)
               ).astype(jnp.bfloat16)
        hi = x.astype(jnp.bfloat16)
        lo = (x - hi.astype(jnp.float32)).astype(jnp.bfloat16)
        o_ref[0] = (
            jax.lax.dot_general(eye, hi, (((1,), (1,)), ((), ())),
                                preferred_element_type=jnp.float32)
            + jax.lax.dot_general(eye, lo, (((1,), (1,)), ((), ())),
                                  preferred_element_type=jnp.float32))
    return body


def _prep_conv_weight(w_mat, cin, cp, cout, r):
    t = w_mat.reshape(3, 3, cin, cout)
    if cin < cp:
        t = jnp.pad(t, ((0, 0), (0, 0), (0, cp - cin), (0, 0)))
    mats = []
    for dy in range(3):
        wm = sum(
            jnp.einsum("pq,cd->pcqd", jnp.eye(r, r, 1 - dx, dtype=w_mat.dtype),
                       t[dy, dx]).reshape(r * cp, r * cout)
            for dx in range(3))
        ml = jnp.zeros((r, r), w_mat.dtype).at[r - 1, 0].set(1.0)
        mr = jnp.zeros((r, r), w_mat.dtype).at[0, r - 1].set(1.0)
        wl = jnp.einsum("pq,cd->pcqd", ml, t[dy, 0]).reshape(r * cp, r * cout)
        wr = jnp.einsum("pq,cd->pcqd", mr, t[dy, 2]).reshape(r * cp, r * cout)
        mats += [wm, wl, wr]
    return jnp.stack(mats)


def _sel_matrices(cp_list):
    pools = []
    for cp in cp_list:
        lo = jnp.arange(128)
        q, c = lo // (2 * cp), lo % (2 * cp)
        valid = c < cp
        li = jnp.arange(128)[:, None]
        pe = (valid & (li == 2 * q * cp + c)).astype(jnp.bfloat16)
        po = (valid & (li == (2 * q + 1) * cp + c)).astype(jnp.bfloat16)
        pools += [pe, po]
    return jnp.stack(pools)


def _fc_kernel(x_ref, w_ref, b_ref, o_ref, acc_ref):
    k = pl.program_id(1)

    @pl.when(k == 0)
    def _():
        acc_ref[...] = jnp.zeros_like(acc_ref)

    acc_ref[...] += jnp.dot(x_ref[...], w_ref[...],
                            preferred_element_type=jnp.float32)

    @pl.when(k == pl.num_programs(1) - 1)
    def _():
        o_ref[...] = (acc_ref[...] + b_ref[...]).astype(o_ref.dtype)


def kernel(x_nchw, conv0_w_mat, conv0_bias, conv0_w_group, conv0_b_group, conv1_w_mat, conv1_bias, conv1_w_group, conv1_b_group, conv2_w_mat, conv2_bias, conv2_w_group, conv2_b_group, conv3_w_mat, conv3_bias, conv3_w_group, conv3_b_group, conv4_w_mat, conv4_bias, conv4_w_group, conv4_b_group, conv5_w_mat, conv5_bias, conv5_w_group, conv5_b_group, conv6_w_mat, conv6_bias, conv6_w_group, conv6_b_group, conv7_w_mat, conv7_bias, conv7_w_group, conv7_b_group, conv8_w_mat, conv8_bias, conv8_w_group, conv8_b_group, conv9_w_mat, conv9_bias, conv9_w_group, conv9_b_group, fc_wt, fc_b):
    w_mats = [conv0_w_mat, conv1_w_mat, conv2_w_mat, conv3_w_mat,
              conv4_w_mat, conv5_w_mat, conv6_w_mat, conv7_w_mat,
              conv8_w_mat, conv9_w_mat]
    b_groups = [conv0_b_group, conv1_b_group, conv2_b_group, conv3_b_group,
                conv4_b_group, conv5_b_group, conv6_b_group, conv7_b_group,
                conv8_b_group, conv9_b_group]

    N, cin0, H0, W0 = x_nchw.shape

    stage_dims = []
    ops = []
    cp_list = []
    H, W = H0, W0
    prev_c = cin0
    for s in range(5):
        la, lb = 2 * s, 2 * s + 1
        if s > 0:
            H, W = H // 2, W // 2
        Ka, cs = w_mats[la].shape
        cin = Ka // 9
        assert cin == prev_c
        r = 128 // cs
        stage_dims.append((H, W, cs))
        cp_list.append(cs)
        ops.append(_prep_conv_weight(w_mats[la], cin, cs, cs, r))
        ops.append(b_groups[la].reshape(1, 128))
        ops.append(_prep_conv_weight(w_mats[lb], cs, cs, cs, r))
        ops.append(b_groups[lb].reshape(1, 128))
        prev_c = cs

    cp0 = cp_list[0]
    li = jnp.arange(W0)[:, None]
    lo = jnp.arange(W0 * cp0)[None, :]
    s_mats = jnp.stack([((lo // cp0 == li) & (lo % cp0 == ch))
                        .astype(jnp.bfloat16) for ch in range(cin0)])
    p_mats = _sel_matrices(cp_list[:4])

    Hf, Wf, cf = stage_dims[-1]
    in_specs = [
        pl.BlockSpec((1, cin0, H0, W0), lambda i: (i, 0, 0, 0)),
        pl.BlockSpec(s_mats.shape, lambda i: (0, 0, 0)),
        pl.BlockSpec(p_mats.shape, lambda i: (0, 0, 0)),
    ]
    for s in range(5):
        in_specs.append(pl.BlockSpec((9, 128, 128), lambda i: (0, 0, 0)))
        in_specs.append(pl.BlockSpec((1, 128), lambda i: (0, 0)))
        in_specs.append(pl.BlockSpec((9, 128, 128), lambda i: (0, 0, 0)))
        in_specs.append(pl.BlockSpec((1, 128), lambda i: (0, 0)))

    feat = pl.pallas_call(
        _make_trunk_kernel(stage_dims, cin0),
        out_shape=jax.ShapeDtypeStruct((N, cf, Hf * Wf), jnp.float32),
        grid_spec=pltpu.PrefetchScalarGridSpec(
            num_scalar_prefetch=0,
            grid=(N,),
            in_specs=in_specs,
            out_specs=pl.BlockSpec((1, cf, Hf * Wf), lambda i: (i, 0, 0)),
        ),
        compiler_params=pltpu.CompilerParams(
            dimension_semantics=("parallel",),
            vmem_limit_bytes=_VMEM_LIMIT),
    )(x_nchw.astype(jnp.float32), s_mats, p_mats, *ops)

    x_flat = feat.reshape(N, cf * Hf * Wf)

    Kfc, Nfc = fc_wt.shape
    tn = 512 if Nfc % 512 == 0 else Nfc
    tk = 1664 if Kfc % 1664 == 0 else Kfc
    out = pl.pallas_call(
        _fc_kernel,
        out_shape=jax.ShapeDtypeStruct((N, Nfc), jnp.float32),
        grid_spec=pltpu.PrefetchScalarGridSpec(
            num_scalar_prefetch=0,
            grid=(Nfc // tn, Kfc // tk),
            in_specs=[
                pl.BlockSpec((N, tk), lambda j, k: (0, k)),
                pl.BlockSpec((tk, tn), lambda j, k: (k, j)),
                pl.BlockSpec((1, tn), lambda j, k: (0, j)),
            ],
            out_specs=pl.BlockSpec((N, tn), lambda j, k: (0, j)),
            scratch_shapes=[pltpu.VMEM((N, tn), jnp.float32)],
        ),
        compiler_params=pltpu.CompilerParams(
            dimension_semantics=("parallel", "arbitrary"),
            vmem_limit_bytes=_VMEM_LIMIT),
    )(x_flat, fc_wt, fc_b.reshape(1, Nfc))
    return out


# speedup vs baseline: 35.0013x; 1.0643x over previous
import jax
import jax.numpy as jnp
from jax.experimental import pallas as pl
from jax.experimental.pallas import tpu as pltpu

_VMEM_LIMIT = 96 * 1024 * 1024


def _split_dot(x, s_bf16):
    hi = x.astype(jnp.bfloat16)
    lo = (x - hi.astype(jnp.float32)).astype(jnp.bfloat16)
    return (jnp.dot(hi, s_bf16, preferred_element_type=jnp.float32)
            + jnp.dot(lo, s_bf16, preferred_element_type=jnp.float32))


def _conv(x, w, b, Mr, chunks):
    rows = jax.lax.broadcasted_iota(jnp.int32, (Mr, 1), 0) % chunks
    not_first = rows != 0
    not_last = rows != (chunks - 1)
    z = jnp.zeros_like(x)
    taps = [
        jnp.pad(x[:-chunks], ((chunks, 0), (0, 0))),
        x,
        jnp.pad(x[chunks:], ((0, chunks), (0, 0))),
    ]
    parts = []
    for dy in range(3):
        a = taps[dy]
        p = jnp.where(not_first, jnp.pad(a[:-1], ((1, 0), (0, 0))), z)
        q = jnp.where(not_last, jnp.pad(a[1:], ((0, 1), (0, 0))), z)
        parts += [a, p, q]
    lhs = jnp.concatenate(parts, axis=1)
    y = jnp.dot(lhs, w.reshape(9 * 128, 128),
                preferred_element_type=jnp.float32)
    return jnp.maximum(y + b, 0.0)


def _pool(x, H, W, cp, pe, po):
    WL = W * cp
    t = x.reshape(H, WL).reshape(H // 2, 2 * WL)
    m = jnp.maximum(t[:, :WL], t[:, WL:])
    mc = m.reshape((H // 2) * (WL // 128), 128)
    return jnp.maximum(_split_dot(mc, pe), _split_dot(mc, po))


def _make_trunk_kernel(stage_dims, cin0):
    def body(x_ref, s_ref, pp_ref, *refs):
        o_ref = refs[-1]
        H0, W0, cp0 = stage_dims[0]
        acc = None
        for ch in range(cin0):
            xi = x_ref[0][ch]
            hi = xi.astype(jnp.bfloat16)
            lo = (xi - hi.astype(jnp.float32)).astype(jnp.bfloat16)
            d = (jnp.dot(hi, s_ref[ch], preferred_element_type=jnp.float32)
                 + jnp.dot(lo, s_ref[ch], preferred_element_type=jnp.float32))
            acc = d if acc is None else acc + d
        chunks = W0 * cp0 // 128
        x = acc.reshape(H0 * chunks, 128)
        for s, (H, W, cp) in enumerate(stage_dims):
            Mr = H * chunks
            x = _conv(x, refs[4 * s][...], refs[4 * s + 1][...], Mr, chunks)
            x = _conv(x, refs[4 * s + 2][...], refs[4 * s + 3][...], Mr, chunks)
            if s < 4:
                x = _pool(x, H, W, cp, pp_ref[2 * s], pp_ref[2 * s + 1])
        eye = (jax.lax.broadcasted_iota(jnp.int32, (128, 128), 0)
               == jax.lax.broadcasted_iota(jnp.int32, (128, 128), 1)
               ).astype(jnp.bfloat16)
        hi = x.astype(jnp.bfloat16)
        lo = (x - hi.astype(jnp.float32)).astype(jnp.bfloat16)
        o_ref[0] = (
            jax.lax.dot_general(eye, hi, (((1,), (1,)), ((), ())),
                                preferred_element_type=jnp.float32)
            + jax.lax.dot_general(eye, lo, (((1,), (1,)), ((), ())),
                                  preferred_element_type=jnp.float32))
    return body


def _prep_conv_weight(w_mat, cin, cp, cout, r):
    t = w_mat.reshape(3, 3, cin, cout)
    if cin < cp:
        t = jnp.pad(t, ((0, 0), (0, 0), (0, cp - cin), (0, 0)))
    mats = []
    for dy in range(3):
        wm = sum(
            jnp.einsum("pq,cd->pcqd", jnp.eye(r, r, 1 - dx, dtype=w_mat.dtype),
                       t[dy, dx]).reshape(r * cp, r * cout)
            for dx in range(3))
        ml = jnp.zeros((r, r), w_mat.dtype).at[r - 1, 0].set(1.0)
        mr = jnp.zeros((r, r), w_mat.dtype).at[0, r - 1].set(1.0)
        wl = jnp.einsum("pq,cd->pcqd", ml, t[dy, 0]).reshape(r * cp, r * cout)
        wr = jnp.einsum("pq,cd->pcqd", mr, t[dy, 2]).reshape(r * cp, r * cout)
        mats += [wm, wl, wr]
    return jnp.stack(mats)


def _sel_matrices(cp_list):
    pools = []
    for cp in cp_list:
        lo = jnp.arange(128)
        q, c = lo // (2 * cp), lo % (2 * cp)
        valid = c < cp
        li = jnp.arange(128)[:, None]
        pe = (valid & (li == 2 * q * cp + c)).astype(jnp.bfloat16)
        po = (valid & (li == (2 * q + 1) * cp + c)).astype(jnp.bfloat16)
        pools += [pe, po]
    return jnp.stack(pools)


def _fc_kernel(x_ref, w_ref, b_ref, o_ref, acc_ref):
    k = pl.program_id(1)

    @pl.when(k == 0)
    def _():
        acc_ref[...] = jnp.zeros_like(acc_ref)

    acc_ref[...] += jnp.dot(x_ref[...], w_ref[...],
                            preferred_element_type=jnp.float32)

    @pl.when(k == pl.num_programs(1) - 1)
    def _():
        o_ref[...] = (acc_ref[...] + b_ref[...]).astype(o_ref.dtype)


def kernel(x_nchw, conv0_w_mat, conv0_bias, conv0_w_group, conv0_b_group, conv1_w_mat, conv1_bias, conv1_w_group, conv1_b_group, conv2_w_mat, conv2_bias, conv2_w_group, conv2_b_group, conv3_w_mat, conv3_bias, conv3_w_group, conv3_b_group, conv4_w_mat, conv4_bias, conv4_w_group, conv4_b_group, conv5_w_mat, conv5_bias, conv5_w_group, conv5_b_group, conv6_w_mat, conv6_bias, conv6_w_group, conv6_b_group, conv7_w_mat, conv7_bias, conv7_w_group, conv7_b_group, conv8_w_mat, conv8_bias, conv8_w_group, conv8_b_group, conv9_w_mat, conv9_bias, conv9_w_group, conv9_b_group, fc_wt, fc_b):
    w_mats = [conv0_w_mat, conv1_w_mat, conv2_w_mat, conv3_w_mat,
              conv4_w_mat, conv5_w_mat, conv6_w_mat, conv7_w_mat,
              conv8_w_mat, conv9_w_mat]
    b_groups = [conv0_b_group, conv1_b_group, conv2_b_group, conv3_b_group,
                conv4_b_group, conv5_b_group, conv6_b_group, conv7_b_group,
                conv8_b_group, conv9_b_group]

    N, cin0, H0, W0 = x_nchw.shape

    stage_dims = []
    ops = []
    cp_list = []
    H, W = H0, W0
    prev_c = cin0
    for s in range(5):
        la, lb = 2 * s, 2 * s + 1
        if s > 0:
            H, W = H // 2, W // 2
        Ka, cs = w_mats[la].shape
        cin = Ka // 9
        assert cin == prev_c
        r = 128 // cs
        stage_dims.append((H, W, cs))
        cp_list.append(cs)
        ops.append(_prep_conv_weight(w_mats[la], cin, cs, cs, r))
        ops.append(b_groups[la].reshape(1, 128))
        ops.append(_prep_conv_weight(w_mats[lb], cs, cs, cs, r))
        ops.append(b_groups[lb].reshape(1, 128))
        prev_c = cs

    cp0 = cp_list[0]
    li = jnp.arange(W0)[:, None]
    lo = jnp.arange(W0 * cp0)[None, :]
    s_mats = jnp.stack([((lo // cp0 == li) & (lo % cp0 == ch))
                        .astype(jnp.bfloat16) for ch in range(cin0)])
    p_mats = _sel_matrices(cp_list[:4])

    Hf, Wf, cf = stage_dims[-1]
    in_specs = [
        pl.BlockSpec((1, cin0, H0, W0), lambda i: (i, 0, 0, 0)),
        pl.BlockSpec(s_mats.shape, lambda i: (0, 0, 0)),
        pl.BlockSpec(p_mats.shape, lambda i: (0, 0, 0)),
    ]
    for s in range(5):
        in_specs.append(pl.BlockSpec((9, 128, 128), lambda i: (0, 0, 0)))
        in_specs.append(pl.BlockSpec((1, 128), lambda i: (0, 0)))
        in_specs.append(pl.BlockSpec((9, 128, 128), lambda i: (0, 0, 0)))
        in_specs.append(pl.BlockSpec((1, 128), lambda i: (0, 0)))

    feat = pl.pallas_call(
        _make_trunk_kernel(stage_dims, cin0),
        out_shape=jax.ShapeDtypeStruct((N, cf, Hf * Wf), jnp.float32),
        grid_spec=pltpu.PrefetchScalarGridSpec(
            num_scalar_prefetch=0,
            grid=(N,),
            in_specs=in_specs,
            out_specs=pl.BlockSpec((1, cf, Hf * Wf), lambda i: (i, 0, 0)),
        ),
        compiler_params=pltpu.CompilerParams(
            dimension_semantics=("parallel",),
            vmem_limit_bytes=_VMEM_LIMIT),
    )(x_nchw.astype(jnp.float32), s_mats, p_mats, *ops)

    x_flat = feat.reshape(N, cf * Hf * Wf)

    Kfc, Nfc = fc_wt.shape
    tn = 512 if Nfc % 512 == 0 else Nfc
    tk = 1664 if Kfc % 1664 == 0 else Kfc
    out = pl.pallas_call(
        _fc_kernel,
        out_shape=jax.ShapeDtypeStruct((N, Nfc), jnp.float32),
        grid_spec=pltpu.PrefetchScalarGridSpec(
            num_scalar_prefetch=0,
            grid=(Nfc // tn, Kfc // tk),
            in_specs=[
                pl.BlockSpec((N, tk), lambda j, k: (0, k)),
                pl.BlockSpec((tk, tn), lambda j, k: (k, j)),
                pl.BlockSpec((1, tn), lambda j, k: (0, j)),
            ],
            out_specs=pl.BlockSpec((N, tn), lambda j, k: (0, j)),
            scratch_shapes=[pltpu.VMEM((N, tn), jnp.float32)],
        ),
        compiler_params=pltpu.CompilerParams(
            dimension_semantics=("parallel", "arbitrary"),
            vmem_limit_bytes=_VMEM_LIMIT),
    )(x_flat, fc_wt, fc_b.reshape(1, Nfc))
    return out
